# baseline (speedup 1.0000x reference)
import jax
import jax.numpy as jnp
from jax import lax
from jax.experimental import pallas as pl
from jax.experimental.pallas import tpu as pltpu

CONF_T = 0.2
IOU_T = 0.6
MAXD = 300
MAXWH = 4096.0
PADD = 512


def _decode_body(pred_ref, anch_ref, st_ref, x1_ref, y1_ref, x2_ref, y2_ref,
                 cf_ref, cl_ref, *, na, nc, rows, w):
    st = st_ref[0, 0]
    pr = pred_ref[0, 0]
    ncls = nc - 5
    riota = lax.broadcasted_iota(jnp.int32, (rows, 128), 0).astype(jnp.float32)
    ciota = lax.broadcasted_iota(jnp.int32, (rows, 128), 1).astype(jnp.float32)
    flat = riota * 128.0 + ciota
    wf = jnp.float32(w)
    gy = jnp.floor(flat / wf)
    gx = flat - gy * wf
    for a in range(na):
        base = nc * a
        sx = jax.nn.sigmoid(pr[base + 0])
        sy = jax.nn.sigmoid(pr[base + 1])
        sw = jax.nn.sigmoid(pr[base + 2])
        sh = jax.nn.sigmoid(pr[base + 3])
        so = jax.nn.sigmoid(pr[base + 4])
        cls_s = jax.nn.sigmoid(pr[base + 5:base + nc]) * so[None]
        conf = jnp.max(cls_s, axis=0)
        rio = lax.broadcasted_iota(jnp.int32, (ncls, rows, 128), 0).astype(jnp.float32)
        clsf = jnp.min(jnp.where(cls_s == conf[None], rio, jnp.float32(1e9)),
                       axis=0)
        aw = anch_ref[0, a, 0]
        ah = anch_ref[0, a, 1]
        cx = (sx * 3.0 - 1.0 + gx) * st
        cy = (sy * 3.0 - 1.0 + gy) * st
        bw = (sw * 2.0) ** 2 * aw
        bh = (sh * 2.0) ** 2 * ah
        sl = slice(rows * a, rows * (a + 1))
        x1_ref[0, 0, sl, :] = cx - bw / 2.0
        y1_ref[0, 0, sl, :] = cy - bh / 2.0
        x2_ref[0, 0, sl, :] = cx + bw / 2.0
        y2_ref[0, 0, sl, :] = cy + bh / 2.0
        cf_ref[0, 0, sl, :] = conf
        cl_ref[0, 0, sl, :] = clsf


def _nms_body(cf_ref, x1_ref, y1_ref, x2_ref, y2_ref, cl_ref, out_ref,
              cscr_ref, *, nrows):
    cm0 = cf_ref[0]
    cm0 = jnp.where(cm0 > CONF_T, cm0, 0.0)
    cscr_ref[:, :] = cm0
    lane128 = lax.broadcasted_iota(jnp.int32, (1, 128), 1)
    lane128f = lane128.astype(jnp.float32)
    lanerow = lax.broadcasted_iota(jnp.int32, (1, nrows), 1)
    lanerowf = lanerow.astype(jnp.float32)
    lane = lax.broadcasted_iota(jnp.int32, (1, PADD), 1)
    z = jnp.zeros((1, PADD), jnp.float32)
    bigf = jnp.float32(1e9)

    ci2 = lax.broadcasted_iota(jnp.int32, (nrows, 128), 1).astype(jnp.float32)
    rowmax = jnp.max(cm0, axis=1)
    rowarg = jnp.min(jnp.where(cm0 == rowmax[:, None], ci2, bigf), axis=1)
    rm0 = rowmax.reshape(1, nrows)
    rc0 = rowarg.reshape(1, nrows)

    def sbody(k, carry):
        rm, rc, sx1, sy1, sx2, sy2, sconf, scls = carry
        m = jnp.max(rm)
        rowm = rm == m
        r = jnp.min(jnp.where(rowm, lanerowf, bigf)).astype(jnp.int32)
        rsel = lanerow == r
        c = jnp.sum(jnp.where(rsel, rc, 0.0)).astype(jnp.int32)
        colm = lane128 == c

        def pick(ref):
            return jnp.sum(jnp.where(colm, ref[0, pl.ds(r, 1), :], 0.0))

        oh = lane == k
        sx1 = jnp.where(oh, pick(x1_ref), sx1)
        sy1 = jnp.where(oh, pick(y1_ref), sy1)
        sx2 = jnp.where(oh, pick(x2_ref), sx2)
        sy2 = jnp.where(oh, pick(y2_ref), sy2)
        scls = jnp.where(oh, pick(cl_ref), scls)
        sconf = jnp.where(oh, m, sconf)
        rowv = cscr_ref[pl.ds(r, 1), :]
        newrow = jnp.where(colm, -1.0, rowv)
        cscr_ref[pl.ds(r, 1), :] = newrow
        nm = jnp.max(newrow)
        nc = jnp.min(jnp.where(newrow == nm, lane128f, bigf))
        rm = jnp.where(rsel, nm, rm)
        rc = jnp.where(rsel, nc, rc)
        return (rm, rc, sx1, sy1, sx2, sy2, sconf, scls)

    _, _, sx1, sy1, sx2, sy2, sconf, scls = lax.fori_loop(
        0, MAXD, sbody, (rm0, rc0, z, z, z, z, z, z))

    keep0 = jnp.where(sconf > CONF_T, 1.0, 0.0)
    off = scls * MAXWH
    ox1 = sx1 + off
    oy1 = sy1 + off
    ox2 = sx2 + off
    oy2 = sy2 + off
    areav = (ox2 - ox1) * (oy2 - oy1)
    lanef = lane.astype(jnp.float32)

    def nbody(i, kk):
        ion = lane == i
        kx1 = jnp.sum(jnp.where(ion, ox1, 0.0))
        ky1 = jnp.sum(jnp.where(ion, oy1, 0.0))
        kx2 = jnp.sum(jnp.where(ion, ox2, 0.0))
        ky2 = jnp.sum(jnp.where(ion, oy2, 0.0))
        ki = jnp.sum(jnp.where(ion, kk, 0.0))
        iw = jnp.maximum(jnp.minimum(kx2, ox2) - jnp.maximum(kx1, ox1), 0.0)
        ih = jnp.maximum(jnp.minimum(ky2, oy2) - jnp.maximum(ky1, oy1), 0.0)
        inter = iw * ih
        ka = (kx2 - kx1) * (ky2 - ky1)
        iou = inter / (ka + areav - inter + 1e-9)
        sup = jnp.where((iou > IOU_T) & (lane > i), 1.0, 0.0) * ki
        return kk * (1.0 - sup)

    kk = lax.fori_loop(0, MAXD, nbody, keep0)

    out_ref[0, 0:1, :] = sx1 * kk
    out_ref[0, 1:2, :] = sy1 * kk
    out_ref[0, 2:3, :] = sx2 * kk
    out_ref[0, 3:4, :] = sy2 * kk
    out_ref[0, 4:5, :] = sconf * kk
    out_ref[0, 5:6, :] = scls * kk
    out_ref[0, 6:7, :] = z
    out_ref[0, 7:8, :] = z


def kernel(preds, anchors, image_size):
    L, B, C, H, W = preds.shape
    NA = anchors.shape[1]
    NC = C // NA
    HW = H * W
    ROWS = HW // 128
    st = jnp.float32(image_size) / jnp.float32(H)
    aeff = (anchors / st) * st
    stm = st.reshape(1, 1)
    preds_r = preds.reshape(L, B, C, ROWS, 128)

    import functools
    fields = pl.pallas_call(
        functools.partial(_decode_body, na=NA, nc=NC, rows=ROWS, w=W),
        grid=(B, L),
        in_specs=[
            pl.BlockSpec((1, 1, C, ROWS, 128), lambda b, i: (i, b, 0, 0, 0)),
            pl.BlockSpec((1, NA, 2), lambda b, i: (i, 0, 0)),
            pl.BlockSpec((1, 1), lambda b, i: (0, 0)),
        ],
        out_specs=[pl.BlockSpec((1, 1, NA * ROWS, 128),
                                lambda b, i: (b, i, 0, 0))] * 6,
        out_shape=[jax.ShapeDtypeStruct((B, L, NA * ROWS, 128), jnp.float32)] * 6,
        compiler_params=pltpu.CompilerParams(
            dimension_semantics=("parallel", "parallel")),
    )(preds_r, aeff, stm)
    x1, y1, x2, y2, cf, cl = [f.reshape(B, L * NA * ROWS, 128) for f in fields]

    NR = L * NA * ROWS
    det = pl.pallas_call(
        functools.partial(_nms_body, nrows=NR),
        grid=(B,),
        in_specs=[pl.BlockSpec((1, NR, 128), lambda b: (b, 0, 0))] * 6,
        out_specs=pl.BlockSpec((1, 8, PADD), lambda b: (b, 0, 0)),
        out_shape=jax.ShapeDtypeStruct((B, 8, PADD), jnp.float32),
        scratch_shapes=[pltpu.VMEM((NR, 128), jnp.float32)],
        compiler_params=pltpu.CompilerParams(
            dimension_semantics=("parallel",)),
    )(cf, x1, y1, x2, y2, cl)

    return det.transpose(0, 2, 1)[:, :MAXD, :6]


# speedup vs baseline: 1.1691x; 1.1691x over previous
import jax
import jax.numpy as jnp
from jax import lax
from jax.experimental import pallas as pl
from jax.experimental.pallas import tpu as pltpu

CONF_T = 0.2
IOU_T = 0.6
MAXD = 300
MAXWH = 4096.0
PADD = 512


def _decode_body(pred_ref, anch_ref, st_ref, x1_ref, y1_ref, x2_ref, y2_ref,
                 cf_ref, cl_ref, *, na, nc, rows, w):
    st = st_ref[0, 0]
    pr = pred_ref[0, 0]
    ncls = nc - 5
    riota = lax.broadcasted_iota(jnp.int32, (rows, 128), 0).astype(jnp.float32)
    ciota = lax.broadcasted_iota(jnp.int32, (rows, 128), 1).astype(jnp.float32)
    flat = riota * 128.0 + ciota
    wf = jnp.float32(w)
    gy = jnp.floor(flat / wf)
    gx = flat - gy * wf
    for a in range(na):
        base = nc * a
        sx = jax.nn.sigmoid(pr[base + 0])
        sy = jax.nn.sigmoid(pr[base + 1])
        sw = jax.nn.sigmoid(pr[base + 2])
        sh = jax.nn.sigmoid(pr[base + 3])
        so = jax.nn.sigmoid(pr[base + 4])
        cls_s = jax.nn.sigmoid(pr[base + 5:base + nc]) * so[None]
        conf = jnp.max(cls_s, axis=0)
        rio = lax.broadcasted_iota(jnp.int32, (ncls, rows, 128), 0).astype(jnp.float32)
        clsf = jnp.min(jnp.where(cls_s == conf[None], rio, jnp.float32(1e9)),
                       axis=0)
        aw = anch_ref[0, a, 0]
        ah = anch_ref[0, a, 1]
        cx = (sx * 3.0 - 1.0 + gx) * st
        cy = (sy * 3.0 - 1.0 + gy) * st
        bw = (sw * 2.0) ** 2 * aw
        bh = (sh * 2.0) ** 2 * ah
        sl = slice(rows * a, rows * (a + 1))
        x1_ref[0, 0, sl, :] = cx - bw / 2.0
        y1_ref[0, 0, sl, :] = cy - bh / 2.0
        x2_ref[0, 0, sl, :] = cx + bw / 2.0
        y2_ref[0, 0, sl, :] = cy + bh / 2.0
        cf_ref[0, 0, sl, :] = conf
        cl_ref[0, 0, sl, :] = clsf


def _nms_body(cf_ref, x1_ref, y1_ref, x2_ref, y2_ref, cl_ref, out_ref,
              cscr_ref, *, nrows):
    cm0 = cf_ref[0]
    cm0 = jnp.where(cm0 > CONF_T, cm0, 0.0)
    cscr_ref[:, :] = cm0
    lane128 = lax.broadcasted_iota(jnp.int32, (1, 128), 1)
    lane128f = lane128.astype(jnp.float32)
    lanerow = lax.broadcasted_iota(jnp.int32, (1, nrows), 1)
    lanerowf = lanerow.astype(jnp.float32)
    lane = lax.broadcasted_iota(jnp.int32, (1, PADD), 1)
    z = jnp.zeros((1, PADD), jnp.float32)
    bigf = jnp.float32(1e9)

    ci2 = lax.broadcasted_iota(jnp.int32, (nrows, 128), 1).astype(jnp.float32)
    rowmax = jnp.max(cm0, axis=1)
    rowarg = jnp.min(jnp.where(cm0 == rowmax[:, None], ci2, bigf), axis=1)
    rm0 = rowmax.reshape(1, nrows)
    rc0 = rowarg.reshape(1, nrows)

    oh0 = jnp.where(lane == 0, 1.0, 0.0)

    def sbody(k, carry):
        rm, rc, oh, sx1, sy1, sx2, sy2, sconf, scls = carry
        m = jnp.max(rm)
        flat = jnp.min(jnp.where(rm == m, lanerowf * 128.0 + rc, bigf))
        rf = jnp.floor(flat * (1.0 / 128.0))
        cf = flat - rf * 128.0
        r = rf.astype(jnp.int32)
        colm = lane128f == cf
        ohb = lane == k

        def pick(ref):
            return jnp.sum(jnp.where(colm, ref[0, pl.ds(r, 1), :], 0.0))

        sx1 = jnp.where(ohb, pick(x1_ref), sx1)
        sy1 = jnp.where(ohb, pick(y1_ref), sy1)
        sx2 = jnp.where(ohb, pick(x2_ref), sx2)
        sy2 = jnp.where(ohb, pick(y2_ref), sy2)
        scls = jnp.where(ohb, pick(cl_ref), scls)
        sconf = jnp.where(ohb, m, sconf)
        rowv = cscr_ref[pl.ds(r, 1), :]
        newrow = jnp.where(colm, -1.0, rowv)
        cscr_ref[pl.ds(r, 1), :] = newrow
        nm = jnp.max(newrow)
        nc = jnp.min(jnp.where(newrow == nm, lane128f, bigf))
        rsel = lanerowf == rf
        rm = jnp.where(rsel, nm, rm)
        rc = jnp.where(rsel, nc, rc)
        return (rm, rc, oh, sx1, sy1, sx2, sy2, sconf, scls)

    _, _, _, sx1, sy1, sx2, sy2, sconf, scls = lax.fori_loop(
        0, MAXD, sbody, (rm0, rc0, oh0, z, z, z, z, z, z))

    keep0 = jnp.where(sconf > CONF_T, 1.0, 0.0)
    off = scls * MAXWH
    ox1 = sx1 + off
    oy1 = sy1 + off
    ox2 = sx2 + off
    oy2 = sy2 + off
    areav = (ox2 - ox1) * (oy2 - oy1)
    gt0 = 1.0 - oh0

    def nbody(i, carry):
        kk, ion, gt = carry
        ionb = lane == i
        kx1 = jnp.sum(jnp.where(ionb, ox1, 0.0))
        ky1 = jnp.sum(jnp.where(ionb, oy1, 0.0))
        kx2 = jnp.sum(jnp.where(ionb, ox2, 0.0))
        ky2 = jnp.sum(jnp.where(ionb, oy2, 0.0))
        ki = jnp.sum(jnp.where(ionb, kk, 0.0))
        iw = jnp.maximum(jnp.minimum(kx2, ox2) - jnp.maximum(kx1, ox1), 0.0)
        ih = jnp.maximum(jnp.minimum(ky2, oy2) - jnp.maximum(ky1, oy1), 0.0)
        inter = iw * ih
        ka = (kx2 - kx1) * (ky2 - ky1)
        iou = inter / (ka + areav - inter + 1e-9)
        sup = jnp.where((iou > IOU_T) & (lane > i), 1.0, 0.0) * ki
        return (kk * (1.0 - sup), ion, gt)

    kk, _, _ = lax.fori_loop(0, MAXD, nbody, (keep0, oh0, gt0))

    out_ref[0, 0:1, :] = sx1 * kk
    out_ref[0, 1:2, :] = sy1 * kk
    out_ref[0, 2:3, :] = sx2 * kk
    out_ref[0, 3:4, :] = sy2 * kk
    out_ref[0, 4:5, :] = sconf * kk
    out_ref[0, 5:6, :] = scls * kk
    out_ref[0, 6:7, :] = z
    out_ref[0, 7:8, :] = z


def kernel(preds, anchors, image_size):
    L, B, C, H, W = preds.shape
    NA = anchors.shape[1]
    NC = C // NA
    HW = H * W
    ROWS = HW // 128
    st = jnp.float32(image_size) / jnp.float32(H)
    aeff = (anchors / st) * st
    stm = st.reshape(1, 1)
    preds_r = preds.reshape(L, B, C, ROWS, 128)

    import functools
    fields = pl.pallas_call(
        functools.partial(_decode_body, na=NA, nc=NC, rows=ROWS, w=W),
        grid=(B, L),
        in_specs=[
            pl.BlockSpec((1, 1, C, ROWS, 128), lambda b, i: (i, b, 0, 0, 0)),
            pl.BlockSpec((1, NA, 2), lambda b, i: (i, 0, 0)),
            pl.BlockSpec((1, 1), lambda b, i: (0, 0)),
        ],
        out_specs=[pl.BlockSpec((1, 1, NA * ROWS, 128),
                                lambda b, i: (b, i, 0, 0))] * 6,
        out_shape=[jax.ShapeDtypeStruct((B, L, NA * ROWS, 128), jnp.float32)] * 6,
        compiler_params=pltpu.CompilerParams(
            dimension_semantics=("parallel", "parallel")),
    )(preds_r, aeff, stm)
    x1, y1, x2, y2, cf, cl = [f.reshape(B, L * NA * ROWS, 128) for f in fields]

    NR = L * NA * ROWS
    det = pl.pallas_call(
        functools.partial(_nms_body, nrows=NR),
        grid=(B,),
        in_specs=[pl.BlockSpec((1, NR, 128), lambda b: (b, 0, 0))] * 6,
        out_specs=pl.BlockSpec((1, 8, PADD), lambda b: (b, 0, 0)),
        out_shape=jax.ShapeDtypeStruct((B, 8, PADD), jnp.float32),
        scratch_shapes=[pltpu.VMEM((NR, 128), jnp.float32)],
        compiler_params=pltpu.CompilerParams(
            dimension_semantics=("parallel",)),
    )(cf, x1, y1, x2, y2, cl)

    return det.transpose(0, 2, 1)[:, :MAXD, :6]


# speedup vs baseline: 1.1714x; 1.0020x over previous
import jax
import jax.numpy as jnp
from jax import lax
from jax.experimental import pallas as pl
from jax.experimental.pallas import tpu as pltpu

CONF_T = 0.2
IOU_T = 0.6
MAXD = 300
MAXWH = 4096.0
PADD = 512


def _decode_body(pred_ref, anch_ref, st_ref, cf_ref, fld_ref, *, na, nc,
                 rows, w):
    st = st_ref[0, 0]
    pr = pred_ref[0, 0]
    ncls = nc - 5
    riota = lax.broadcasted_iota(jnp.int32, (rows, 128), 0).astype(jnp.float32)
    ciota = lax.broadcasted_iota(jnp.int32, (rows, 128), 1).astype(jnp.float32)
    flat = riota * 128.0 + ciota
    wf = jnp.float32(w)
    gy = jnp.floor(flat / wf)
    gx = flat - gy * wf
    for a in range(na):
        base = nc * a
        sx = jax.nn.sigmoid(pr[base + 0])
        sy = jax.nn.sigmoid(pr[base + 1])
        sw = jax.nn.sigmoid(pr[base + 2])
        sh = jax.nn.sigmoid(pr[base + 3])
        so = jax.nn.sigmoid(pr[base + 4])
        cls_s = jax.nn.sigmoid(pr[base + 5:base + nc]) * so[None]
        conf = jnp.max(cls_s, axis=0)
        rio = lax.broadcasted_iota(jnp.int32, (ncls, rows, 128), 0).astype(jnp.float32)
        clsf = jnp.min(jnp.where(cls_s == conf[None], rio, jnp.float32(1e9)),
                       axis=0)
        aw = anch_ref[0, a, 0]
        ah = anch_ref[0, a, 1]
        cx = (sx * 3.0 - 1.0 + gx) * st
        cy = (sy * 3.0 - 1.0 + gy) * st
        bw = (sw * 2.0) ** 2 * aw
        bh = (sh * 2.0) ** 2 * ah
        sl = slice(rows * a, rows * (a + 1))
        fld_ref[0, 0, sl, 0:128] = cx - bw / 2.0
        fld_ref[0, 0, sl, 128:256] = cy - bh / 2.0
        fld_ref[0, 0, sl, 256:384] = cx + bw / 2.0
        fld_ref[0, 0, sl, 384:512] = cy + bh / 2.0
        fld_ref[0, 0, sl, 512:640] = clsf
        cf_ref[0, 0, sl, :] = conf


def _nms_body(cf_ref, fld_ref, out_ref, cscr_ref, *, nrows):
    cm0 = cf_ref[0]
    cm0 = jnp.where(cm0 > CONF_T, cm0, 0.0)
    cscr_ref[:, :] = cm0
    lane128 = lax.broadcasted_iota(jnp.int32, (1, 128), 1)
    lane128f = lane128.astype(jnp.float32)
    lanerow = lax.broadcasted_iota(jnp.int32, (1, nrows), 1)
    lanerowf = lanerow.astype(jnp.float32)
    lane = lax.broadcasted_iota(jnp.int32, (1, PADD), 1)
    z = jnp.zeros((1, PADD), jnp.float32)
    bigf = jnp.float32(1e9)

    ci2 = lax.broadcasted_iota(jnp.int32, (nrows, 128), 1).astype(jnp.float32)
    rowmax = jnp.max(cm0, axis=1)
    rowarg = jnp.min(jnp.where(cm0 == rowmax[:, None], ci2, bigf), axis=1)
    rm0 = rowmax.reshape(1, nrows)
    rc0 = rowarg.reshape(1, nrows)

    def sbody(k, carry):
        rm, rc, sflat, sconf = carry
        m = jnp.max(rm)
        flat = jnp.min(jnp.where(rm == m, lanerowf * 128.0 + rc, bigf))
        rf = jnp.floor(flat * (1.0 / 128.0))
        cf = flat - rf * 128.0
        r = rf.astype(jnp.int32)
        colm = lane128f == cf
        ohb = lane == k
        sflat = jnp.where(ohb, flat, sflat)
        sconf = jnp.where(ohb, m, sconf)
        rowv = cscr_ref[pl.ds(r, 1), :]
        newrow = jnp.where(colm, -1.0, rowv)
        cscr_ref[pl.ds(r, 1), :] = newrow
        nm = jnp.max(newrow)
        nc = jnp.min(jnp.where(newrow == nm, lane128f, bigf))
        rsel = lanerowf == rf
        rm = jnp.where(rsel, nm, rm)
        rc = jnp.where(rsel, nc, rc)
        return (rm, rc, sflat, sconf)

    _, _, sflat, sconf = lax.fori_loop(
        0, MAXD, sbody, (rm0, rc0, z, z))

    flatT = sflat.reshape(PADD, 1)
    rK = jnp.floor(flatT * (1.0 / 128.0))
    cK = flatT - rK * 128.0
    iotaR = lax.broadcasted_iota(jnp.int32, (PADD, nrows), 1).astype(jnp.float32)
    ohR = jnp.where(iotaR == rK, 1.0, 0.0)
    G = jnp.dot(ohR, fld_ref[0], preferred_element_type=jnp.float32)
    iotaC = lax.broadcasted_iota(jnp.int32, (PADD, 128), 1).astype(jnp.float32)
    ohC = jnp.where(iotaC == cK, 1.0, 0.0)
    vals = jnp.sum(G.reshape(PADD, 5, 128) * ohC[:, None, :], axis=2)
    valsT = vals.T
    sx1 = valsT[0:1, :]
    sy1 = valsT[1:2, :]
    sx2 = valsT[2:3, :]
    sy2 = valsT[3:4, :]
    scls = valsT[4:5, :]

    keep0 = jnp.where(sconf > CONF_T, 1.0, 0.0)
    off = scls * MAXWH
    ox1 = sx1 + off
    oy1 = sy1 + off
    ox2 = sx2 + off
    oy2 = sy2 + off
    areav = (ox2 - ox1) * (oy2 - oy1)

    def nbody(i, kk):
        ionb = lane == i
        kx1 = jnp.sum(jnp.where(ionb, ox1, 0.0))
        ky1 = jnp.sum(jnp.where(ionb, oy1, 0.0))
        kx2 = jnp.sum(jnp.where(ionb, ox2, 0.0))
        ky2 = jnp.sum(jnp.where(ionb, oy2, 0.0))
        ki = jnp.sum(jnp.where(ionb, kk, 0.0))
        iw = jnp.maximum(jnp.minimum(kx2, ox2) - jnp.maximum(kx1, ox1), 0.0)
        ih = jnp.maximum(jnp.minimum(ky2, oy2) - jnp.maximum(ky1, oy1), 0.0)
        inter = iw * ih
        ka = (kx2 - kx1) * (ky2 - ky1)
        iou = inter / (ka + areav - inter + 1e-9)
        sup = jnp.where((iou > IOU_T) & (lane > i), 1.0, 0.0) * ki
        return kk * (1.0 - sup)

    kk = lax.fori_loop(0, MAXD, nbody, keep0)

    out_ref[0, 0:1, :] = sx1 * kk
    out_ref[0, 1:2, :] = sy1 * kk
    out_ref[0, 2:3, :] = sx2 * kk
    out_ref[0, 3:4, :] = sy2 * kk
    out_ref[0, 4:5, :] = sconf * kk
    out_ref[0, 5:6, :] = scls * kk
    out_ref[0, 6:7, :] = z
    out_ref[0, 7:8, :] = z


def kernel(preds, anchors, image_size):
    L, B, C, H, W = preds.shape
    NA = anchors.shape[1]
    NC = C // NA
    HW = H * W
    ROWS = HW // 128
    st = jnp.float32(image_size) / jnp.float32(H)
    aeff = (anchors / st) * st
    stm = st.reshape(1, 1)
    preds_r = preds.reshape(L, B, C, ROWS, 128)

    import functools
    fields = pl.pallas_call(
        functools.partial(_decode_body, na=NA, nc=NC, rows=ROWS, w=W),
        grid=(B, L),
        in_specs=[
            pl.BlockSpec((1, 1, C, ROWS, 128), lambda b, i: (i, b, 0, 0, 0)),
            pl.BlockSpec((1, NA, 2), lambda b, i: (i, 0, 0)),
            pl.BlockSpec((1, 1), lambda b, i: (0, 0)),
        ],
        out_specs=[pl.BlockSpec((1, 1, NA * ROWS, 128),
                                lambda b, i: (b, i, 0, 0)),
                   pl.BlockSpec((1, 1, NA * ROWS, 640),
                                lambda b, i: (b, i, 0, 0))],
        out_shape=[jax.ShapeDtypeStruct((B, L, NA * ROWS, 128), jnp.float32),
                   jax.ShapeDtypeStruct((B, L, NA * ROWS, 640), jnp.float32)],
        compiler_params=pltpu.CompilerParams(
            dimension_semantics=("parallel", "parallel")),
    )(preds_r, aeff, stm)
    NR = L * NA * ROWS
    cf = fields[0].reshape(B, NR, 128)
    fld = fields[1].reshape(B, NR, 640)

    det = pl.pallas_call(
        functools.partial(_nms_body, nrows=NR),
        grid=(B,),
        in_specs=[pl.BlockSpec((1, NR, 128), lambda b: (b, 0, 0)),
                  pl.BlockSpec((1, NR, 640), lambda b: (b, 0, 0))],
        out_specs=pl.BlockSpec((1, 8, PADD), lambda b: (b, 0, 0)),
        out_shape=jax.ShapeDtypeStruct((B, 8, PADD), jnp.float32),
        scratch_shapes=[pltpu.VMEM((NR, 128), jnp.float32)],
        compiler_params=pltpu.CompilerParams(
            dimension_semantics=("parallel",)),
    )(cf, fld)

    return det.transpose(0, 2, 1)[:, :MAXD, :6]


# speedup vs baseline: 1.3661x; 1.1662x over previous
import jax
import jax.numpy as jnp
from jax import lax
from jax.experimental import pallas as pl
from jax.experimental.pallas import tpu as pltpu

CONF_T = 0.2
IOU_T = 0.6
MAXD = 300
MAXWH = 4096.0
PADD = 512


def _decode_body(pred_ref, anch_ref, st_ref, cf_ref, fld_ref, *, na, nc,
                 rows, w):
    st = st_ref[0, 0]
    pr = pred_ref[0, 0]
    ncls = nc - 5
    riota = lax.broadcasted_iota(jnp.int32, (rows, 128), 0).astype(jnp.float32)
    ciota = lax.broadcasted_iota(jnp.int32, (rows, 128), 1).astype(jnp.float32)
    flat = riota * 128.0 + ciota
    wf = jnp.float32(w)
    gy = jnp.floor(flat / wf)
    gx = flat - gy * wf
    for a in range(na):
        base = nc * a
        sx = jax.nn.sigmoid(pr[base + 0])
        sy = jax.nn.sigmoid(pr[base + 1])
        sw = jax.nn.sigmoid(pr[base + 2])
        sh = jax.nn.sigmoid(pr[base + 3])
        so = jax.nn.sigmoid(pr[base + 4])
        cls_s = jax.nn.sigmoid(pr[base + 5:base + nc]) * so[None]
        conf = jnp.max(cls_s, axis=0)
        rio = lax.broadcasted_iota(jnp.int32, (ncls, rows, 128), 0).astype(jnp.float32)
        clsf = jnp.min(jnp.where(cls_s == conf[None], rio, jnp.float32(1e9)),
                       axis=0)
        aw = anch_ref[0, a, 0]
        ah = anch_ref[0, a, 1]
        cx = (sx * 3.0 - 1.0 + gx) * st
        cy = (sy * 3.0 - 1.0 + gy) * st
        bw = (sw * 2.0) ** 2 * aw
        bh = (sh * 2.0) ** 2 * ah
        sl = slice(rows * a, rows * (a + 1))
        fld_ref[0, 0, sl, 0:128] = cx - bw / 2.0
        fld_ref[0, 0, sl, 128:256] = cy - bh / 2.0
        fld_ref[0, 0, sl, 256:384] = cx + bw / 2.0
        fld_ref[0, 0, sl, 384:512] = cy + bh / 2.0
        fld_ref[0, 0, sl, 512:640] = clsf
        cf_ref[0, 0, sl, :] = conf


def _nms_body(cf_ref, fld_ref, out_ref, cscr_ref, *, nrows):
    cm0 = cf_ref[0]
    cm0 = jnp.where(cm0 > CONF_T, cm0, 0.0)
    cscr_ref[:, :] = cm0
    lane128 = lax.broadcasted_iota(jnp.int32, (1, 128), 1)
    lane128f = lane128.astype(jnp.float32)
    lanerow = lax.broadcasted_iota(jnp.int32, (1, nrows), 1)
    lanerowf = lanerow.astype(jnp.float32)
    lane = lax.broadcasted_iota(jnp.int32, (1, PADD), 1)
    z = jnp.zeros((1, PADD), jnp.float32)
    bigf = jnp.float32(1e9)

    ci2 = lax.broadcasted_iota(jnp.int32, (nrows, 128), 1).astype(jnp.float32)
    rowmax = jnp.max(cm0, axis=1)
    rowarg = jnp.min(jnp.where(cm0 == rowmax[:, None], ci2, bigf), axis=1)
    rm0 = rowmax.reshape(1, nrows)
    rc0 = rowarg.reshape(1, nrows)

    def sbody(k, carry):
        rm, rc, sflat, sconf = carry
        m = jnp.max(rm)
        flat = jnp.min(jnp.where(rm == m, lanerowf * 128.0 + rc, bigf))
        rf = jnp.floor(flat * (1.0 / 128.0))
        cf = flat - rf * 128.0
        r = rf.astype(jnp.int32)
        colm = lane128f == cf
        ohb = lane == k
        sflat = jnp.where(ohb, flat, sflat)
        sconf = jnp.where(ohb, m, sconf)
        rowv = cscr_ref[pl.ds(r, 1), :]
        newrow = jnp.where(colm, -1.0, rowv)
        cscr_ref[pl.ds(r, 1), :] = newrow
        nm = jnp.max(newrow)
        nc = jnp.min(jnp.where(newrow == nm, lane128f, bigf))
        rsel = lanerowf == rf
        rm = jnp.where(rsel, nm, rm)
        rc = jnp.where(rsel, nc, rc)
        return (rm, rc, sflat, sconf)

    _, _, sflat, sconf = lax.fori_loop(
        0, MAXD, sbody, (rm0, rc0, z, z), unroll=4)

    flatT = sflat.reshape(PADD, 1)
    rK = jnp.floor(flatT * (1.0 / 128.0))
    cK = flatT - rK * 128.0
    iotaR = lax.broadcasted_iota(jnp.int32, (PADD, nrows), 1).astype(jnp.float32)
    ohR = jnp.where(iotaR == rK, 1.0, 0.0)
    G = jnp.dot(ohR, fld_ref[0], preferred_element_type=jnp.float32,
                precision=jax.lax.Precision.HIGHEST)
    iotaC = lax.broadcasted_iota(jnp.int32, (PADD, 128), 1).astype(jnp.float32)
    ohC = jnp.where(iotaC == cK, 1.0, 0.0)
    vals = jnp.sum(G.reshape(PADD, 5, 128) * ohC[:, None, :], axis=2)
    valsT = vals.T
    sx1 = valsT[0:1, :]
    sy1 = valsT[1:2, :]
    sx2 = valsT[2:3, :]
    sy2 = valsT[3:4, :]
    scls = valsT[4:5, :]

    keep0 = jnp.where(sconf > CONF_T, 1.0, 0.0)
    off = scls * MAXWH
    ox1 = sx1 + off
    oy1 = sy1 + off
    ox2 = sx2 + off
    oy2 = sy2 + off
    areav = (ox2 - ox1) * (oy2 - oy1)

    def nbody(i, kk):
        ionb = lane == i
        kx1 = jnp.sum(jnp.where(ionb, ox1, 0.0))
        ky1 = jnp.sum(jnp.where(ionb, oy1, 0.0))
        kx2 = jnp.sum(jnp.where(ionb, ox2, 0.0))
        ky2 = jnp.sum(jnp.where(ionb, oy2, 0.0))
        ki = jnp.sum(jnp.where(ionb, kk, 0.0))
        iw = jnp.maximum(jnp.minimum(kx2, ox2) - jnp.maximum(kx1, ox1), 0.0)
        ih = jnp.maximum(jnp.minimum(ky2, oy2) - jnp.maximum(ky1, oy1), 0.0)
        inter = iw * ih
        ka = (kx2 - kx1) * (ky2 - ky1)
        iou = inter / (ka + areav - inter + 1e-9)
        sup = jnp.where((iou > IOU_T) & (lane > i), 1.0, 0.0) * ki
        return kk * (1.0 - sup)

    kk = lax.fori_loop(0, MAXD, nbody, keep0, unroll=4)

    out_ref[0, 0:1, :] = sx1 * kk
    out_ref[0, 1:2, :] = sy1 * kk
    out_ref[0, 2:3, :] = sx2 * kk
    out_ref[0, 3:4, :] = sy2 * kk
    out_ref[0, 4:5, :] = sconf * kk
    out_ref[0, 5:6, :] = scls * kk
    out_ref[0, 6:7, :] = z
    out_ref[0, 7:8, :] = z


def kernel(preds, anchors, image_size):
    L, B, C, H, W = preds.shape
    NA = anchors.shape[1]
    NC = C // NA
    HW = H * W
    ROWS = HW // 128
    st = jnp.float32(image_size) / jnp.float32(H)
    aeff = (anchors / st) * st
    stm = st.reshape(1, 1)
    preds_r = preds.reshape(L, B, C, ROWS, 128)

    import functools
    fields = pl.pallas_call(
        functools.partial(_decode_body, na=NA, nc=NC, rows=ROWS, w=W),
        grid=(B, L),
        in_specs=[
            pl.BlockSpec((1, 1, C, ROWS, 128), lambda b, i: (i, b, 0, 0, 0)),
            pl.BlockSpec((1, NA, 2), lambda b, i: (i, 0, 0)),
            pl.BlockSpec((1, 1), lambda b, i: (0, 0)),
        ],
        out_specs=[pl.BlockSpec((1, 1, NA * ROWS, 128),
                                lambda b, i: (b, i, 0, 0)),
                   pl.BlockSpec((1, 1, NA * ROWS, 640),
                                lambda b, i: (b, i, 0, 0))],
        out_shape=[jax.ShapeDtypeStruct((B, L, NA * ROWS, 128), jnp.float32),
                   jax.ShapeDtypeStruct((B, L, NA * ROWS, 640), jnp.float32)],
        compiler_params=pltpu.CompilerParams(
            dimension_semantics=("parallel", "parallel")),
    )(preds_r, aeff, stm)
    NR = L * NA * ROWS
    cf = fields[0].reshape(B, NR, 128)
    fld = fields[1].reshape(B, NR, 640)

    det = pl.pallas_call(
        functools.partial(_nms_body, nrows=NR),
        grid=(B,),
        in_specs=[pl.BlockSpec((1, NR, 128), lambda b: (b, 0, 0)),
                  pl.BlockSpec((1, NR, 640), lambda b: (b, 0, 0))],
        out_specs=pl.BlockSpec((1, 8, PADD), lambda b: (b, 0, 0)),
        out_shape=jax.ShapeDtypeStruct((B, 8, PADD), jnp.float32),
        scratch_shapes=[pltpu.VMEM((NR, 128), jnp.float32)],
        compiler_params=pltpu.CompilerParams(
            dimension_semantics=("parallel",)),
    )(cf, fld)

    return det.transpose(0, 2, 1)[:, :MAXD, :6]


# speedup vs baseline: 1.4114x; 1.0331x over previous
import jax
import jax.numpy as jnp
from jax import lax
from jax.experimental import pallas as pl
from jax.experimental.pallas import tpu as pltpu

CONF_T = 0.2
IOU_T = 0.6
MAXD = 300
MAXWH = 4096.0
PADD = 512


def _decode_body(pred_ref, anch_ref, st_ref, cf_ref, fld_ref, *, na, nc,
                 rows, w):
    st = st_ref[0, 0]
    pr = pred_ref[0, 0]
    ncls = nc - 5
    riota = lax.broadcasted_iota(jnp.int32, (rows, 128), 0).astype(jnp.float32)
    ciota = lax.broadcasted_iota(jnp.int32, (rows, 128), 1).astype(jnp.float32)
    flat = riota * 128.0 + ciota
    wf = jnp.float32(w)
    gy = jnp.floor(flat / wf)
    gx = flat - gy * wf
    for a in range(na):
        base = nc * a
        sx = jax.nn.sigmoid(pr[base + 0])
        sy = jax.nn.sigmoid(pr[base + 1])
        sw = jax.nn.sigmoid(pr[base + 2])
        sh = jax.nn.sigmoid(pr[base + 3])
        so = jax.nn.sigmoid(pr[base + 4])
        cls_s = jax.nn.sigmoid(pr[base + 5:base + nc]) * so[None]
        conf = jnp.max(cls_s, axis=0)
        rio = lax.broadcasted_iota(jnp.int32, (ncls, rows, 128), 0).astype(jnp.float32)
        clsf = jnp.min(jnp.where(cls_s == conf[None], rio, jnp.float32(1e9)),
                       axis=0)
        aw = anch_ref[0, a, 0]
        ah = anch_ref[0, a, 1]
        cx = (sx * 3.0 - 1.0 + gx) * st
        cy = (sy * 3.0 - 1.0 + gy) * st
        bw = (sw * 2.0) ** 2 * aw
        bh = (sh * 2.0) ** 2 * ah
        sl = slice(rows * a, rows * (a + 1))
        fld_ref[0, 0, sl, 0:128] = cx - bw / 2.0
        fld_ref[0, 0, sl, 128:256] = cy - bh / 2.0
        fld_ref[0, 0, sl, 256:384] = cx + bw / 2.0
        fld_ref[0, 0, sl, 384:512] = cy + bh / 2.0
        fld_ref[0, 0, sl, 512:640] = clsf
        cf_ref[0, 0, sl, :] = conf


def _nms_body(cf_ref, fld_ref, out_ref, cscr_ref, *, nrows):
    cm0 = cf_ref[0]
    cm0 = jnp.where(cm0 > CONF_T, cm0, 0.0)
    cscr_ref[:, :] = cm0
    lane128 = lax.broadcasted_iota(jnp.int32, (1, 128), 1)
    lane128f = lane128.astype(jnp.float32)
    lanerow = lax.broadcasted_iota(jnp.int32, (1, nrows), 1)
    lanerowf = lanerow.astype(jnp.float32)
    lane = lax.broadcasted_iota(jnp.int32, (1, PADD), 1)
    z = jnp.zeros((1, PADD), jnp.float32)
    bigf = jnp.float32(1e9)

    ci2 = lax.broadcasted_iota(jnp.int32, (nrows, 128), 1).astype(jnp.float32)
    rowmax = jnp.max(cm0, axis=1)
    rowarg = jnp.min(jnp.where(cm0 == rowmax[:, None], ci2, bigf), axis=1)
    rm0 = rowmax.reshape(1, nrows)
    rc0 = rowarg.reshape(1, nrows)

    def sbody(k, carry):
        rm, rc, sflat, sconf = carry
        m = jnp.max(rm)
        flat = jnp.min(jnp.where(rm == m, lanerowf * 128.0 + rc, bigf))
        rf = jnp.floor(flat * (1.0 / 128.0))
        cf = flat - rf * 128.0
        r = rf.astype(jnp.int32)
        colm = lane128f == cf
        ohb = lane == k
        sflat = jnp.where(ohb, flat, sflat)
        sconf = jnp.where(ohb, m, sconf)
        rowv = cscr_ref[pl.ds(r, 1), :]
        newrow = jnp.where(colm, -1.0, rowv)
        cscr_ref[pl.ds(r, 1), :] = newrow
        nm = jnp.max(newrow)
        nc = jnp.min(jnp.where(newrow == nm, lane128f, bigf))
        rsel = lanerowf == rf
        rm = jnp.where(rsel, nm, rm)
        rc = jnp.where(rsel, nc, rc)
        return (rm, rc, sflat, sconf)

    _, _, sflat, sconf = lax.fori_loop(
        0, MAXD, sbody, (rm0, rc0, z, z), unroll=8)

    flatT = sflat.reshape(PADD, 1)
    rK = jnp.floor(flatT * (1.0 / 128.0))
    cK = flatT - rK * 128.0
    iotaR = lax.broadcasted_iota(jnp.int32, (PADD, nrows), 1).astype(jnp.float32)
    ohR = jnp.where(iotaR == rK, 1.0, 0.0)
    G = jnp.dot(ohR, fld_ref[0], preferred_element_type=jnp.float32,
                precision=jax.lax.Precision.HIGHEST)
    iotaC = lax.broadcasted_iota(jnp.int32, (PADD, 128), 1).astype(jnp.float32)
    ohC = jnp.where(iotaC == cK, 1.0, 0.0)
    vals = jnp.sum(G.reshape(PADD, 5, 128) * ohC[:, None, :], axis=2)
    valsT = vals.T
    sx1 = valsT[0:1, :]
    sy1 = valsT[1:2, :]
    sx2 = valsT[2:3, :]
    sy2 = valsT[3:4, :]
    scls = valsT[4:5, :]

    keep0 = jnp.where(sconf > CONF_T, 1.0, 0.0)
    off = scls * MAXWH
    ox1 = sx1 + off
    oy1 = sy1 + off
    ox2 = sx2 + off
    oy2 = sy2 + off
    areav = (ox2 - ox1) * (oy2 - oy1)

    def nbody(i, kk):
        ionb = lane == i
        kx1 = jnp.sum(jnp.where(ionb, ox1, 0.0))
        ky1 = jnp.sum(jnp.where(ionb, oy1, 0.0))
        kx2 = jnp.sum(jnp.where(ionb, ox2, 0.0))
        ky2 = jnp.sum(jnp.where(ionb, oy2, 0.0))
        ki = jnp.sum(jnp.where(ionb, kk, 0.0))
        iw = jnp.maximum(jnp.minimum(kx2, ox2) - jnp.maximum(kx1, ox1), 0.0)
        ih = jnp.maximum(jnp.minimum(ky2, oy2) - jnp.maximum(ky1, oy1), 0.0)
        inter = iw * ih
        ka = (kx2 - kx1) * (ky2 - ky1)
        iou = inter / (ka + areav - inter + 1e-9)
        sup = jnp.where((iou > IOU_T) & (lane > i), 1.0, 0.0) * ki
        return kk * (1.0 - sup)

    kk = lax.fori_loop(0, MAXD, nbody, keep0, unroll=8)

    out_ref[0, 0:1, :] = sx1 * kk
    out_ref[0, 1:2, :] = sy1 * kk
    out_ref[0, 2:3, :] = sx2 * kk
    out_ref[0, 3:4, :] = sy2 * kk
    out_ref[0, 4:5, :] = sconf * kk
    out_ref[0, 5:6, :] = scls * kk
    out_ref[0, 6:7, :] = z
    out_ref[0, 7:8, :] = z


def kernel(preds, anchors, image_size):
    L, B, C, H, W = preds.shape
    NA = anchors.shape[1]
    NC = C // NA
    HW = H * W
    ROWS = HW // 128
    st = jnp.float32(image_size) / jnp.float32(H)
    aeff = (anchors / st) * st
    stm = st.reshape(1, 1)
    preds_r = preds.reshape(L, B, C, ROWS, 128)

    import functools
    fields = pl.pallas_call(
        functools.partial(_decode_body, na=NA, nc=NC, rows=ROWS, w=W),
        grid=(B, L),
        in_specs=[
            pl.BlockSpec((1, 1, C, ROWS, 128), lambda b, i: (i, b, 0, 0, 0)),
            pl.BlockSpec((1, NA, 2), lambda b, i: (i, 0, 0)),
            pl.BlockSpec((1, 1), lambda b, i: (0, 0)),
        ],
        out_specs=[pl.BlockSpec((1, 1, NA * ROWS, 128),
                                lambda b, i: (b, i, 0, 0)),
                   pl.BlockSpec((1, 1, NA * ROWS, 640),
                                lambda b, i: (b, i, 0, 0))],
        out_shape=[jax.ShapeDtypeStruct((B, L, NA * ROWS, 128), jnp.float32),
                   jax.ShapeDtypeStruct((B, L, NA * ROWS, 640), jnp.float32)],
        compiler_params=pltpu.CompilerParams(
            dimension_semantics=("parallel", "parallel")),
    )(preds_r, aeff, stm)
    NR = L * NA * ROWS
    cf = fields[0].reshape(B, NR, 128)
    fld = fields[1].reshape(B, NR, 640)

    det = pl.pallas_call(
        functools.partial(_nms_body, nrows=NR),
        grid=(B,),
        in_specs=[pl.BlockSpec((1, NR, 128), lambda b: (b, 0, 0)),
                  pl.BlockSpec((1, NR, 640), lambda b: (b, 0, 0))],
        out_specs=pl.BlockSpec((1, 8, PADD), lambda b: (b, 0, 0)),
        out_shape=jax.ShapeDtypeStruct((B, 8, PADD), jnp.float32),
        scratch_shapes=[pltpu.VMEM((NR, 128), jnp.float32)],
        compiler_params=pltpu.CompilerParams(
            dimension_semantics=("parallel",)),
    )(cf, fld)

    return det.transpose(0, 2, 1)[:, :MAXD, :6]


# speedup vs baseline: 1.4353x; 1.0169x over previous
import jax
import jax.numpy as jnp
from jax import lax
from jax.experimental import pallas as pl
from jax.experimental.pallas import tpu as pltpu

CONF_T = 0.2
IOU_T = 0.6
MAXD = 300
MAXWH = 4096.0
PADD = 512


def _decode_body(pred_ref, anch_ref, st_ref, cf_ref, fld_ref, *, na, nc,
                 rows, w):
    st = st_ref[0, 0]
    pr = pred_ref[0, 0]
    ncls = nc - 5
    riota = lax.broadcasted_iota(jnp.int32, (rows, 128), 0).astype(jnp.float32)
    ciota = lax.broadcasted_iota(jnp.int32, (rows, 128), 1).astype(jnp.float32)
    flat = riota * 128.0 + ciota
    wf = jnp.float32(w)
    gy = jnp.floor(flat / wf)
    gx = flat - gy * wf
    for a in range(na):
        base = nc * a
        sx = jax.nn.sigmoid(pr[base + 0])
        sy = jax.nn.sigmoid(pr[base + 1])
        sw = jax.nn.sigmoid(pr[base + 2])
        sh = jax.nn.sigmoid(pr[base + 3])
        so = jax.nn.sigmoid(pr[base + 4])
        cls_s = jax.nn.sigmoid(pr[base + 5:base + nc]) * so[None]
        conf = jnp.max(cls_s, axis=0)
        rio = lax.broadcasted_iota(jnp.int32, (ncls, rows, 128), 0).astype(jnp.float32)
        clsf = jnp.min(jnp.where(cls_s == conf[None], rio, jnp.float32(1e9)),
                       axis=0)
        aw = anch_ref[0, a, 0]
        ah = anch_ref[0, a, 1]
        cx = (sx * 3.0 - 1.0 + gx) * st
        cy = (sy * 3.0 - 1.0 + gy) * st
        bw = (sw * 2.0) ** 2 * aw
        bh = (sh * 2.0) ** 2 * ah
        sl = slice(rows * a, rows * (a + 1))
        fld_ref[0, 0, sl, 0:128] = cx - bw / 2.0
        fld_ref[0, 0, sl, 128:256] = cy - bh / 2.0
        fld_ref[0, 0, sl, 256:384] = cx + bw / 2.0
        fld_ref[0, 0, sl, 384:512] = cy + bh / 2.0
        fld_ref[0, 0, sl, 512:640] = clsf
        cf_ref[0, 0, sl, :] = conf


def _nms_body(cf_ref, fld_ref, out_ref, cscr_ref, *, nrows):
    cm0 = cf_ref[0]
    cm0 = jnp.where(cm0 > CONF_T, cm0, 0.0)
    cscr_ref[:, :] = cm0
    lane128 = lax.broadcasted_iota(jnp.int32, (1, 128), 1)
    lane128f = lane128.astype(jnp.float32)
    lanerow = lax.broadcasted_iota(jnp.int32, (1, nrows), 1)
    lanerowf = lanerow.astype(jnp.float32)
    lane = lax.broadcasted_iota(jnp.int32, (1, PADD), 1)
    z = jnp.zeros((1, PADD), jnp.float32)
    bigf = jnp.float32(1e9)

    ci2 = lax.broadcasted_iota(jnp.int32, (nrows, 128), 1).astype(jnp.float32)
    rowmax = jnp.max(cm0, axis=1)
    rowarg = jnp.min(jnp.where(cm0 == rowmax[:, None], ci2, bigf), axis=1)
    rm0 = rowmax.reshape(1, nrows)
    rc0 = rowarg.reshape(1, nrows)

    def sbody(k, carry):
        rm, rc, sflat, sconf = carry
        m = jnp.max(rm)
        flat = jnp.min(jnp.where(rm == m, lanerowf * 128.0 + rc, bigf))
        rf = jnp.floor(flat * (1.0 / 128.0))
        cf = flat - rf * 128.0
        r = rf.astype(jnp.int32)
        colm = lane128f == cf
        ohb = lane == k
        sflat = jnp.where(ohb, flat, sflat)
        sconf = jnp.where(ohb, m, sconf)
        rowv = cscr_ref[pl.ds(r, 1), :]
        newrow = jnp.where(colm, -1.0, rowv)
        cscr_ref[pl.ds(r, 1), :] = newrow
        nm = jnp.max(newrow)
        nc = jnp.min(jnp.where(newrow == nm, lane128f, bigf))
        rsel = lanerowf == rf
        rm = jnp.where(rsel, nm, rm)
        rc = jnp.where(rsel, nc, rc)
        return (rm, rc, sflat, sconf)

    _, _, sflat, sconf = lax.fori_loop(
        0, MAXD, sbody, (rm0, rc0, z, z), unroll=16)

    flatT = sflat.reshape(PADD, 1)
    rK = jnp.floor(flatT * (1.0 / 128.0))
    cK = flatT - rK * 128.0
    iotaR = lax.broadcasted_iota(jnp.int32, (PADD, nrows), 1).astype(jnp.float32)
    ohR = jnp.where(iotaR == rK, 1.0, 0.0)
    G = jnp.dot(ohR, fld_ref[0], preferred_element_type=jnp.float32,
                precision=jax.lax.Precision.HIGHEST)
    iotaC = lax.broadcasted_iota(jnp.int32, (PADD, 128), 1).astype(jnp.float32)
    ohC = jnp.where(iotaC == cK, 1.0, 0.0)
    vals = jnp.sum(G.reshape(PADD, 5, 128) * ohC[:, None, :], axis=2)
    valsT = vals.T
    sx1 = valsT[0:1, :]
    sy1 = valsT[1:2, :]
    sx2 = valsT[2:3, :]
    sy2 = valsT[3:4, :]
    scls = valsT[4:5, :]

    keep0 = jnp.where(sconf > CONF_T, 1.0, 0.0)
    off = scls * MAXWH
    ox1 = sx1 + off
    oy1 = sy1 + off
    ox2 = sx2 + off
    oy2 = sy2 + off
    areav = (ox2 - ox1) * (oy2 - oy1)

    def nbody(i, kk):
        ionb = lane == i
        kx1 = jnp.sum(jnp.where(ionb, ox1, 0.0))
        ky1 = jnp.sum(jnp.where(ionb, oy1, 0.0))
        kx2 = jnp.sum(jnp.where(ionb, ox2, 0.0))
        ky2 = jnp.sum(jnp.where(ionb, oy2, 0.0))
        ki = jnp.sum(jnp.where(ionb, kk, 0.0))
        iw = jnp.maximum(jnp.minimum(kx2, ox2) - jnp.maximum(kx1, ox1), 0.0)
        ih = jnp.maximum(jnp.minimum(ky2, oy2) - jnp.maximum(ky1, oy1), 0.0)
        inter = iw * ih
        ka = (kx2 - kx1) * (ky2 - ky1)
        iou = inter / (ka + areav - inter + 1e-9)
        sup = jnp.where((iou > IOU_T) & (lane > i), 1.0, 0.0) * ki
        return kk * (1.0 - sup)

    kk = lax.fori_loop(0, MAXD, nbody, keep0, unroll=16)

    out_ref[0, 0:1, :] = sx1 * kk
    out_ref[0, 1:2, :] = sy1 * kk
    out_ref[0, 2:3, :] = sx2 * kk
    out_ref[0, 3:4, :] = sy2 * kk
    out_ref[0, 4:5, :] = sconf * kk
    out_ref[0, 5:6, :] = scls * kk
    out_ref[0, 6:7, :] = z
    out_ref[0, 7:8, :] = z


def kernel(preds, anchors, image_size):
    L, B, C, H, W = preds.shape
    NA = anchors.shape[1]
    NC = C // NA
    HW = H * W
    ROWS = HW // 128
    st = jnp.float32(image_size) / jnp.float32(H)
    aeff = (anchors / st) * st
    stm = st.reshape(1, 1)
    preds_r = preds.reshape(L, B, C, ROWS, 128)

    import functools
    fields = pl.pallas_call(
        functools.partial(_decode_body, na=NA, nc=NC, rows=ROWS, w=W),
        grid=(B, L),
        in_specs=[
            pl.BlockSpec((1, 1, C, ROWS, 128), lambda b, i: (i, b, 0, 0, 0)),
            pl.BlockSpec((1, NA, 2), lambda b, i: (i, 0, 0)),
            pl.BlockSpec((1, 1), lambda b, i: (0, 0)),
        ],
        out_specs=[pl.BlockSpec((1, 1, NA * ROWS, 128),
                                lambda b, i: (b, i, 0, 0)),
                   pl.BlockSpec((1, 1, NA * ROWS, 640),
                                lambda b, i: (b, i, 0, 0))],
        out_shape=[jax.ShapeDtypeStruct((B, L, NA * ROWS, 128), jnp.float32),
                   jax.ShapeDtypeStruct((B, L, NA * ROWS, 640), jnp.float32)],
        compiler_params=pltpu.CompilerParams(
            dimension_semantics=("parallel", "parallel")),
    )(preds_r, aeff, stm)
    NR = L * NA * ROWS
    cf = fields[0].reshape(B, NR, 128)
    fld = fields[1].reshape(B, NR, 640)

    det = pl.pallas_call(
        functools.partial(_nms_body, nrows=NR),
        grid=(B,),
        in_specs=[pl.BlockSpec((1, NR, 128), lambda b: (b, 0, 0)),
                  pl.BlockSpec((1, NR, 640), lambda b: (b, 0, 0))],
        out_specs=pl.BlockSpec((1, 8, PADD), lambda b: (b, 0, 0)),
        out_shape=jax.ShapeDtypeStruct((B, 8, PADD), jnp.float32),
        scratch_shapes=[pltpu.VMEM((NR, 128), jnp.float32)],
        compiler_params=pltpu.CompilerParams(
            dimension_semantics=("parallel",)),
    )(cf, fld)

    return det.transpose(0, 2, 1)[:, :MAXD, :6]
